# Initial kernel scaffold; baseline (speedup 1.0000x reference)
import jax
import jax.numpy as jnp
from jax.experimental import pallas as pl


def kernel(x, edge_index, edge_list, Wq, bq, Wk, bk, shift):
    raise NotImplementedError("write your pallas kernel here")


# speedup vs baseline: 30.2530x; 30.2530x over previous
import functools

import jax
import jax.numpy as jnp
from jax import lax
from jax.experimental import pallas as pl
from jax.experimental.pallas import tpu as pltpu
from jax.experimental.pallas import tpu_sc as plsc

NC = 2
NS = 16
NW = NC * NS


def _proj_body(nl, x_ref, wq_ref, bq_ref, wk_ref, bk_ref, s_ref,
               q_ref, k_ref, y_ref):
    xb = x_ref[...]
    q_ref[...] = jnp.dot(xb, wq_ref[...],
                         preferred_element_type=jnp.float32) + bq_ref[...]
    k_ref[...] = jnp.dot(xb, wk_ref[...],
                         preferred_element_type=jnp.float32) + bk_ref[...]
    for l in range(nl):
        y_ref[l] = jnp.dot(xb, s_ref[l], preferred_element_type=jnp.float32)


def _combine_body(m0_ref, m1_ref, d0_ref, d1_ref, o_ref):
    m = m0_ref[...] + m1_ref[...]
    dd = d0_ref[...] + d1_ref[...]
    o_ref[...] = m / jnp.where(dd == 0.0, 1.0, dd)


def _sc_body(n_y, n_pad, d, co, go, c_sz, nchunk, epw, npt,
             q_hbm, k_hbm, y_hbm, src_hbm, dst_hbm, lst_hbm,
             om_hbm, od_hbm,
             src_v, dst_v, yidx_v, dgrp_v, qrows, krows, msg, den_stage,
             acc, den_sh, sem_q, sem_k, sem_y):
    cid = lax.axis_index("c")
    sid = lax.axis_index("s")
    wid = sid * NC + cid
    inv_s = jnp.float32(1.0 / (go ** 0.5))
    zeros16 = jnp.zeros((16,), jnp.float32)
    lane = lax.iota(jnp.int32, 16)
    xors = [jnp.bitwise_xor(lane, 1 << b) for b in range(4)]
    tb = sid * npt
    db = tb // 8

    def _zm(i, c_):
        msg[i // 8, pl.ds((i % 8) * 16, 16)] = zeros16
        den_stage[i // 8, pl.ds((i % 8) * 16, 16)] = zeros16
        return c_

    lax.fori_loop(0, c_sz * 8, _zm, 0)

    def _za(i, c_):
        pltpu.sync_copy(msg, acc.at[pl.ds(tb + i * c_sz, c_sz)])
        return c_

    lax.fori_loop(0, npt // c_sz, _za, 0)
    pltpu.sync_copy(msg, den_sh.at[pl.ds(db, c_sz)])
    pltpu.sync_copy(msg.at[pl.ds(0, npt // 8 - c_sz)],
                    den_sh.at[pl.ds(db + c_sz, npt // 8 - c_sz)])
    plsc.subcore_barrier()

    ebase = wid * epw

    def _chunk(ci, carry):
        base = ebase + ci * c_sz
        pltpu.sync_copy(src_hbm.at[pl.ds(base, c_sz)], src_v)
        pltpu.sync_copy(dst_hbm.at[pl.ds(base, c_sz)], dst_v)
        pltpu.sync_copy(lst_hbm.at[pl.ds(base, c_sz)], yidx_v)

        def _yi(j, c2):
            sl = pl.ds(j * 16, 16)
            yidx_v[sl] = yidx_v[sl] * n_y + src_v[sl]
            dgrp_v[sl] = lax.shift_right_logical(dst_v[sl], 3)
            return c2

        lax.fori_loop(0, c_sz // 16, _yi, 0)

        cq = pltpu.async_copy(q_hbm.at[dst_v], qrows, sem_q)
        ck = pltpu.async_copy(k_hbm.at[src_v], krows, sem_k)
        cy = pltpu.async_copy(y_hbm.at[yidx_v], msg, sem_y)
        cq.wait()
        ck.wait()
        cy.wait()

        def _grp(g, c2):
            dvec = dst_v[pl.ds(g * 16, 16)]
            for i in range(16):
                j = g * 16 + i
                tail = zeros16
                for c in range(co):
                    p = (qrows[j, pl.ds(c * go, 16)]
                         * krows[j, pl.ds(c * go, 16)]
                         + qrows[j, pl.ds(c * go + 16, 16)]
                         * krows[j, pl.ds(c * go + 16, 16)])
                    for xm in xors:
                        p = p + p.at[xm].get(mode="promise_in_bounds")
                    ev = jnp.exp(p * inv_s)
                    msg[j, pl.ds(c * go, 16)] = (
                        msg[j, pl.ds(c * go, 16)] * ev)
                    msg[j, pl.ds(c * go + 16, 16)] = (
                        msg[j, pl.ds(c * go + 16, 16)] * ev)
                    tail = jnp.where(lane == c, ev, tail)
                off = jnp.bitwise_and(dvec[i], 7) * 16
                den_stage[j, pl.ds(off, 16)] = tail
            return c2

        lax.fori_loop(0, c_sz // 16, _grp, 0)

        pltpu.sync_copy(msg, acc.at[dst_v], add=True)
        pltpu.sync_copy(den_stage, den_sh.at[dgrp_v], add=True)

        def _cln(g, c2):
            dvec = dst_v[pl.ds(g * 16, 16)]
            for i in range(16):
                off = jnp.bitwise_and(dvec[i], 7) * 16
                den_stage[g * 16 + i, pl.ds(off, 16)] = zeros16
            return c2

        lax.fori_loop(0, c_sz // 16, _cln, 0)
        return carry

    lax.fori_loop(0, nchunk, _chunk, 0)
    plsc.subcore_barrier()

    ob = cid * n_pad + tb
    pltpu.sync_copy(acc.at[pl.ds(tb, npt)], om_hbm.at[pl.ds(ob, npt)])

    def _exp(ch, c_):
        pltpu.sync_copy(den_sh.at[pl.ds(db + ch * (c_sz // 8), c_sz // 8)],
                        den_stage.at[pl.ds(0, c_sz // 8)])

        def _nd(no, c2):
            vals = den_stage[lax.shift_right_logical(no, 3),
                             pl.ds(jnp.bitwise_and(no, 7) * 16, 16)]
            for c in range(co):
                v = jnp.full((16,), vals[c], jnp.float32)
                msg[no, pl.ds(c * go, 16)] = v
                msg[no, pl.ds(c * go + 16, 16)] = v
            return c2

        lax.fori_loop(0, c_sz, _nd, 0)
        pltpu.sync_copy(msg, od_hbm.at[pl.ds(ob + ch * c_sz, c_sz)])
        return c_

    lax.fori_loop(0, npt // c_sz, _exp, 0)


def kernel(x, edge_index, edge_list, Wq, bq, Wk, bk, shift):
    n, co, go = x.shape
    d = co * go
    e = edge_index.shape[1]
    nl = shift.shape[0]
    n_pad = 10240
    npt = n_pad // NS
    c_sz = 64
    nchunk = -(-e // (NW * c_sz))
    epw = nchunk * c_sz
    e_pad = epw * NW - e

    xf = x.reshape(n, d)
    src = jnp.concatenate([edge_index[0], jnp.zeros(e_pad, jnp.int32)])
    dst = jnp.concatenate(
        [edge_index[1], jnp.full(e_pad, n_pad - 1, jnp.int32)])
    elist = jnp.concatenate([edge_list, jnp.zeros(e_pad, jnp.int32)])

    eye = jnp.eye(co, dtype=jnp.float32)
    S = (eye[None, :, None, :, None]
         * shift[:, None, :, None, :]).reshape(nl, d, d)

    b1 = 400
    q, k, Y = pl.pallas_call(
        functools.partial(_proj_body, nl),
        grid=(n // b1,),
        in_specs=[
            pl.BlockSpec((b1, d), lambda i: (i, 0)),
            pl.BlockSpec((d, d), lambda i: (0, 0)),
            pl.BlockSpec((1, d), lambda i: (0, 0)),
            pl.BlockSpec((d, d), lambda i: (0, 0)),
            pl.BlockSpec((1, d), lambda i: (0, 0)),
            pl.BlockSpec((nl, d, d), lambda i: (0, 0, 0)),
        ],
        out_specs=[
            pl.BlockSpec((b1, d), lambda i: (i, 0)),
            pl.BlockSpec((b1, d), lambda i: (i, 0)),
            pl.BlockSpec((nl, b1, d), lambda i: (0, i, 0)),
        ],
        out_shape=[
            jax.ShapeDtypeStruct((n, d), jnp.float32),
            jax.ShapeDtypeStruct((n, d), jnp.float32),
            jax.ShapeDtypeStruct((nl, n, d), jnp.float32),
        ],
    )(xf, Wq, bq.reshape(1, d), Wk, bk.reshape(1, d), S)
    yf = Y.reshape(nl * n, d)

    q = jnp.pad(q, ((0, n_pad - n), (0, 0)))
    sc_fn = pl.kernel(
        functools.partial(_sc_body, n, n_pad, d, co, go, c_sz, nchunk,
                          epw, npt),
        out_type=(
            jax.ShapeDtypeStruct((2 * n_pad, d), jnp.float32),
            jax.ShapeDtypeStruct((2 * n_pad, d), jnp.float32),
        ),
        mesh=plsc.VectorSubcoreMesh(core_axis_name="c", subcore_axis_name="s",
                                    num_cores=NC, num_subcores=NS),
        compiler_params=pltpu.CompilerParams(needs_layout_passes=False),
        scratch_types=[
            pltpu.VMEM((c_sz,), jnp.int32),
            pltpu.VMEM((c_sz,), jnp.int32),
            pltpu.VMEM((c_sz,), jnp.int32),
            pltpu.VMEM((c_sz,), jnp.int32),
            pltpu.VMEM((c_sz, d), jnp.float32),
            pltpu.VMEM((c_sz, d), jnp.float32),
            pltpu.VMEM((c_sz, d), jnp.float32),
            pltpu.VMEM((c_sz, d), jnp.float32),
            pltpu.VMEM_SHARED((n_pad, d), jnp.float32),
            pltpu.VMEM_SHARED((n_pad // 8, d), jnp.float32),
            pltpu.SemaphoreType.DMA,
            pltpu.SemaphoreType.DMA,
            pltpu.SemaphoreType.DMA,
        ],
    )
    om, od = sc_fn(q, k, yf, src, dst, elist)

    b2 = 1280
    nb2 = n_pad // b2
    out = pl.pallas_call(
        _combine_body,
        grid=(nb2,),
        in_specs=[
            pl.BlockSpec((b2, d), lambda i: (i, 0)),
            pl.BlockSpec((b2, d), lambda i: (i + nb2, 0)),
            pl.BlockSpec((b2, d), lambda i: (i, 0)),
            pl.BlockSpec((b2, d), lambda i: (i + nb2, 0)),
        ],
        out_specs=pl.BlockSpec((b2, d), lambda i: (i, 0)),
        out_shape=jax.ShapeDtypeStruct((n_pad, d), jnp.float32),
    )(om, om, od, od)

    return out[:n].reshape(n, co, go)
